# baseline (speedup 1.0000x reference)
import functools

import numpy as np
import jax
import jax.numpy as jnp
from jax import lax
from jax.experimental import pallas as pl
from jax.experimental.pallas import tpu as pltpu
from jax.experimental.pallas import tpu_sc as plsc

_C = 128
_R = 5


def _sc_label_gather(lab, typ, stk, V):
    N = lab.shape[0]
    H = stk.shape[1]
    zidx = 3 * V
    mesh = plsc.VectorSubcoreMesh(
        core_axis_name="c", subcore_axis_name="s", num_cores=2, num_subcores=16
    )
    nw = 32
    per_w = N // nw
    n_chunks = per_w // _C
    n_waves = n_chunks // _R

    @functools.partial(
        pl.kernel,
        out_type=jax.ShapeDtypeStruct((N, H), jnp.float32),
        mesh=mesh,
        compiler_params=pltpu.CompilerParams(
            use_tc_tiling_on_sc=False, disable_bounds_checks=True),
        scratch_types=[
            pltpu.VMEM((per_w,), jnp.int32),
            pltpu.VMEM((per_w,), jnp.int32),
        ]
        + [pltpu.VMEM((_C,), jnp.int32) for _ in range(_R)]
        + [pltpu.VMEM((_C, H), jnp.float32) for _ in range(_R)]
        + [pltpu.SemaphoreType.DMA, pltpu.SemaphoreType.DMA],
    )
    def k(lab_hbm, typ_hbm, stk_hbm, out_hbm, lab_v, typ_v, *rest):
        idx_bufs = rest[:_R]
        row_bufs = rest[_R:2 * _R]
        gsem, wsem = rest[2 * _R], rest[2 * _R + 1]
        wid = lax.axis_index("s") * 2 + lax.axis_index("c")
        base = wid * per_w
        pltpu.sync_copy(lab_hbm.at[pl.ds(base, per_w)], lab_v)
        pltpu.sync_copy(typ_hbm.at[pl.ds(base, per_w)], typ_v)

        def wave(w, carry):
            g0 = w * _R
            gathers = []
            for b in range(_R):
                tok = (g0 + b) * _C
                for r in range(_C // 16):
                    sl_src = pl.ds(tok + r * 16, 16)
                    sl_dst = pl.ds(r * 16, 16)
                    t = typ_v[sl_src]
                    l = lab_v[sl_src]
                    z = jnp.full((16,), zidx, jnp.int32)
                    idx_bufs[b][sl_dst] = jnp.where(t == 0, z, l + (t - 1) * V)
                gathers.append(
                    pltpu.async_copy(stk_hbm.at[idx_bufs[b]], row_bufs[b], gsem))
            for cp in gathers:
                cp.wait()
            writes = []
            for b in range(_R):
                tok = (g0 + b) * _C
                writes.append(
                    pltpu.async_copy(row_bufs[b], out_hbm.at[pl.ds(base + tok, _C)],
                                     wsem))
            for cp in writes:
                cp.wait()
            return carry

        lax.fori_loop(0, n_waves, wave, 0)

    return k(lab, typ, stk)


def _tc_dense(gath, vals, times, poss, ons, age_i, gen_i,
              W1, b1, W2, b2, lw, lb, pw, pb, posg, posp,
              on_tab, age_tab, gen_tab, task_tab, g_, beta_):
    B, S, H = gath.shape
    BB = 8
    G = B // BB
    OV = on_tab.shape[0]
    NA = age_tab.shape[0]

    def body(gath_ref, val_ref, time_ref, pos_ref, on_ref,
             age_ref, gen_ref, W1_ref, b1_ref, W2_ref, b2_ref,
             lw_ref, lb_ref, pw_ref, pb_ref, posg_ref, posp_ref,
             ontab_ref, agetab_ref, gentab_ref, tasktab_ref,
             g_ref, beta_ref, out_ref):
        def norm(x):
            mu = jnp.mean(x, axis=-1, keepdims=True)
            var = jnp.mean((x - mu) ** 2, axis=-1, keepdims=True)
            return (x - mu) / jnp.sqrt(var + 1e-12) * g_ref[...] + beta_ref[...]

        vT = jnp.transpose(val_ref[...])
        tT = jnp.transpose(time_ref[...])
        pT = jnp.transpose(pos_ref[...])
        oT = jnp.transpose(on_ref[...])
        lane = lax.broadcasted_iota(jnp.int32, (S, H), 1)
        ohiota = lax.broadcasted_iota(jnp.int32, (S, OV), 1).astype(jnp.float32)
        task_n = norm(tasktab_ref[...]).reshape(1, 1, H)
        out_ref[:, 0:1, :] = jnp.broadcast_to(task_n, (BB, 1, H))
        for r in range(BB):
            rows = gath_ref[r]
            v = vT[:, r:r + 1]
            h1 = jnp.maximum(v * W1_ref[...] + b1_ref[...], 0.0)
            ve = jnp.dot(h1, W2_ref[...],
                         preferred_element_type=jnp.float32) + b2_ref[...]
            tt = tT[:, r:r + 1]
            te = jnp.where(lane == 0, tt * lw_ref[...] + lb_ref[...],
                           jnp.sin(tt * pw_ref[...] + pb_ref[...]))
            pf = pT[:, r:r + 1]
            pe = jnp.sin(pf * posg_ref[...] + posp_ref[...])
            oid = oT[:, r:r + 1]
            oh = (ohiota == oid).astype(jnp.float32)
            oe = jnp.dot(oh, ontab_ref[...], preferred_element_type=jnp.float32)
            emb = rows + ve + te + pe + oe
            out_ref[r, 3:, :] = norm(emb)
            aid = age_ref[r, 0]
            am = lax.broadcasted_iota(jnp.int32, (NA, H), 0) == aid
            ae = jnp.sum(jnp.where(am, agetab_ref[...], 0.0), axis=0, keepdims=True)
            gid = gen_ref[r, 0]
            gm = lax.broadcasted_iota(jnp.int32, (2, H), 0) == gid
            ge = jnp.sum(jnp.where(gm, gentab_ref[...], 0.0), axis=0, keepdims=True)
            out_ref[r, 1:2, :] = norm(ae)
            out_ref[r, 2:3, :] = norm(ge)

    const = lambda shape: pl.BlockSpec(shape, lambda i: tuple(0 for _ in shape))
    return pl.pallas_call(
        body,
        grid=(G,),
        in_specs=[
            pl.BlockSpec((BB, S, H), lambda i: (i, 0, 0)),
            pl.BlockSpec((BB, S), lambda i: (i, 0)),
            pl.BlockSpec((BB, S), lambda i: (i, 0)),
            pl.BlockSpec((BB, S), lambda i: (i, 0)),
            pl.BlockSpec((BB, S), lambda i: (i, 0)),
            pl.BlockSpec((BB, 1), lambda i: (i, 0), memory_space=pltpu.SMEM),
            pl.BlockSpec((BB, 1), lambda i: (i, 0), memory_space=pltpu.SMEM),
            const((1, H)),
            const((1, H)),
            const((H, H)),
            const((1, H)),
            const((1, 1)),
            const((1, 1)),
            const((1, H)),
            const((1, H)),
            const((1, H)),
            const((1, H)),
            const((OV, H)),
            const((NA, H)),
            const((2, H)),
            const((1, H)),
            const((1, H)),
            const((1, H)),
        ],
        out_specs=pl.BlockSpec((BB, 3 + S, H), lambda i: (i, 0, 0)),
        out_shape=jax.ShapeDtypeStruct((B, 3 + S, H), jnp.float32),
    )(gath, vals, times, poss, ons, age_i, gen_i, W1, b1, W2, b2,
      lw, lb, pw, pb, posg, posp, on_tab, age_tab, gen_tab, task_tab, g_, beta_)


def kernel(label_ids, value_ids, time_ids, on_ids, position_ids, token_type,
           age_ids, gender_ids, task_token, proc_table, med_table, chart_table,
           W1, b1, W2, b2, t2v_lw, t2v_lb, t2v_pw, t2v_pb,
           on_table, pos_table, age_table, gender_table, task_table, ln_g, ln_b):
    B, S = label_ids.shape
    H = proc_table.shape[1]
    V = proc_table.shape[0]
    N = B * S
    lab = label_ids.reshape(N).astype(jnp.int32)
    typ = token_type.reshape(N).astype(jnp.int32)
    stk = jnp.concatenate(
        [proc_table, med_table, chart_table, jnp.zeros((8, H), jnp.float32)], axis=0)
    gath = _sc_label_gather(lab, typ, stk, V).reshape(B, S, H)
    pw_pad = jnp.concatenate([jnp.zeros((1, 1), jnp.float32), t2v_pw], axis=1)
    pb_pad = jnp.concatenate([jnp.zeros((1,), jnp.float32), t2v_pb]).reshape(1, H)
    on_pad = jnp.concatenate(
        [on_table, jnp.zeros((16 - on_table.shape[0], H), jnp.float32)], axis=0)
    j = np.arange(H, dtype=np.float64)
    posg = jnp.asarray((10000.0 ** (-2.0 * j / H)).astype(np.float32)).reshape(1, H)
    posp = jnp.asarray(
        np.where(j % 2 == 0, 0.0, np.pi / 2).astype(np.float32)).reshape(1, H)
    return _tc_dense(
        gath,
        value_ids,
        time_ids,
        position_ids.astype(jnp.float32),
        on_ids.astype(jnp.float32),
        age_ids.astype(jnp.int32),
        gender_ids.astype(jnp.int32),
        W1, b1.reshape(1, H), W2, b2.reshape(1, H),
        t2v_lw, t2v_lb.reshape(1, 1), pw_pad, pb_pad, posg, posp,
        on_pad, age_table, gender_table, task_table,
        ln_g.reshape(1, H), ln_b.reshape(1, H))


# speedup vs baseline: 1.1298x; 1.1298x over previous
import functools

import numpy as np
import jax
import jax.numpy as jnp
from jax import lax
from jax.experimental import pallas as pl
from jax.experimental.pallas import tpu as pltpu
from jax.experimental.pallas import tpu_sc as plsc

_C = 128
_R = 5


def _sc_label_gather(lab, typ, stk, V):
    N = lab.shape[0]
    H = stk.shape[1]
    zidx = 3 * V
    mesh = plsc.VectorSubcoreMesh(
        core_axis_name="c", subcore_axis_name="s", num_cores=2, num_subcores=16
    )
    nw = 32
    per_w = N // nw
    n_chunks = per_w // _C
    n_waves = n_chunks // _R

    @functools.partial(
        pl.kernel,
        out_type=jax.ShapeDtypeStruct((N, H), jnp.float32),
        mesh=mesh,
        compiler_params=pltpu.CompilerParams(
            use_tc_tiling_on_sc=False, disable_bounds_checks=True),
        scratch_types=[
            pltpu.VMEM((per_w,), jnp.int32),
            pltpu.VMEM((per_w,), jnp.int32),
        ]
        + [pltpu.VMEM((_C,), jnp.int32) for _ in range(_R)]
        + [pltpu.VMEM((_C, H), jnp.float32) for _ in range(_R)]
        + [pltpu.SemaphoreType.DMA, pltpu.SemaphoreType.DMA],
    )
    def k(lab_hbm, typ_hbm, stk_hbm, out_hbm, lab_v, typ_v, *rest):
        idx_bufs = rest[:_R]
        row_bufs = rest[_R:2 * _R]
        gsem, wsem = rest[2 * _R], rest[2 * _R + 1]
        wid = lax.axis_index("s") * 2 + lax.axis_index("c")
        base = wid * per_w
        pltpu.sync_copy(lab_hbm.at[pl.ds(base, per_w)], lab_v)
        pltpu.sync_copy(typ_hbm.at[pl.ds(base, per_w)], typ_v)

        def wave(w, carry):
            g0 = w * _R
            gathers = []
            for b in range(_R):
                tok = (g0 + b) * _C
                for r in range(_C // 16):
                    sl_src = pl.ds(tok + r * 16, 16)
                    sl_dst = pl.ds(r * 16, 16)
                    t = typ_v[sl_src]
                    l = lab_v[sl_src]
                    z = jnp.full((16,), zidx, jnp.int32)
                    idx_bufs[b][sl_dst] = jnp.where(t == 0, z, l + (t - 1) * V)
                gathers.append(
                    pltpu.async_copy(stk_hbm.at[idx_bufs[b]], row_bufs[b], gsem))
            for cp in gathers:
                cp.wait()
            writes = []
            for b in range(_R):
                tok = (g0 + b) * _C
                writes.append(
                    pltpu.async_copy(row_bufs[b], out_hbm.at[pl.ds(base + tok, _C)],
                                     wsem))
            for cp in writes:
                cp.wait()
            return carry

        lax.fori_loop(0, n_waves, wave, 0)

    return k(lab, typ, stk)


def _tc_dense(gath, vals, times, poss, ons, age_i, gen_i,
              W1, b1, W2, b2, lw, lb, pw2, pb2, pos_tab,
              on_tab, age_tab, gen_tab, task_tab, g_, beta_):
    B, S, H = gath.shape
    BB = 8
    G = B // BB
    OV = on_tab.shape[0]
    NA = age_tab.shape[0]
    P = pos_tab.shape[0]

    def body(gath_ref, val_ref, time_ref, pos_ref, on_ref,
             age_ref, gen_ref, W1_ref, b1_ref, W2_ref, b2_ref,
             lw_ref, lb_ref, pw2_ref, pb2_ref, postab_ref,
             ontab_ref, agetab_ref, gentab_ref, tasktab_ref,
             g_ref, beta_ref, out_ref):
        def norm(x):
            mu = jnp.mean(x, axis=-1, keepdims=True)
            var = jnp.mean((x - mu) ** 2, axis=-1, keepdims=True)
            return (x - mu) / jnp.sqrt(var + 1e-12) * g_ref[...] + beta_ref[...]

        vT = jnp.transpose(val_ref[...])
        tT = jnp.transpose(time_ref[...])
        pT = jnp.transpose(pos_ref[...])
        oT = jnp.transpose(on_ref[...])
        lane = lax.broadcasted_iota(jnp.int32, (S, H), 1)
        lane2 = lax.broadcasted_iota(jnp.int32, (S, 2 * H), 1)
        ohiota = lax.broadcasted_iota(jnp.int32, (S, OV), 1).astype(jnp.float32)
        phiota = lax.broadcasted_iota(jnp.int32, (S, P), 1).astype(jnp.float32)
        task_n = norm(tasktab_ref[...]).reshape(1, 1, H)
        out_ref[:, 0:1, :] = jnp.broadcast_to(task_n, (BB, 1, H))
        tes = []
        for r in range(0, BB, 2):
            tt0 = tT[:, r:r + 1]
            tt1 = tT[:, r + 1:r + 2]
            tt2 = jnp.where(lane2 < H, tt0, tt1)
            sv = jnp.sin(tt2 * pw2_ref[...] + pb2_ref[...])
            tes.append((tt0, sv[:, 0:H]))
            tes.append((tt1, sv[:, H:2 * H]))
        for r in range(BB):
            rows = gath_ref[r]
            v = vT[:, r:r + 1]
            h1 = jnp.maximum(v * W1_ref[...] + b1_ref[...], 0.0)
            ve = jnp.dot(h1, W2_ref[...],
                         preferred_element_type=jnp.float32) + b2_ref[...]
            tt, tsin = tes[r]
            te = jnp.where(lane == 0, tt * lw_ref[...] + lb_ref[...], tsin)
            ph = (phiota == pT[:, r:r + 1]).astype(jnp.float32)
            pe = jnp.dot(ph, postab_ref[...], preferred_element_type=jnp.float32)
            oh = (ohiota == oT[:, r:r + 1]).astype(jnp.float32)
            oe = jnp.dot(oh, ontab_ref[...], preferred_element_type=jnp.float32)
            emb = rows + ve + te + pe + oe
            out_ref[r, 3:, :] = norm(emb)
            aid = age_ref[r, 0]
            am = lax.broadcasted_iota(jnp.int32, (NA, H), 0) == aid
            ae = jnp.sum(jnp.where(am, agetab_ref[...], 0.0), axis=0, keepdims=True)
            gid = gen_ref[r, 0]
            gm = lax.broadcasted_iota(jnp.int32, (2, H), 0) == gid
            ge = jnp.sum(jnp.where(gm, gentab_ref[...], 0.0), axis=0, keepdims=True)
            out_ref[r, 1:2, :] = norm(ae)
            out_ref[r, 2:3, :] = norm(ge)

    const = lambda shape: pl.BlockSpec(shape, lambda i: tuple(0 for _ in shape))
    return pl.pallas_call(
        body,
        grid=(G,),
        in_specs=[
            pl.BlockSpec((BB, S, H), lambda i: (i, 0, 0)),
            pl.BlockSpec((BB, S), lambda i: (i, 0)),
            pl.BlockSpec((BB, S), lambda i: (i, 0)),
            pl.BlockSpec((BB, S), lambda i: (i, 0)),
            pl.BlockSpec((BB, S), lambda i: (i, 0)),
            pl.BlockSpec((BB, 1), lambda i: (i, 0), memory_space=pltpu.SMEM),
            pl.BlockSpec((BB, 1), lambda i: (i, 0), memory_space=pltpu.SMEM),
            const((1, H)),
            const((1, H)),
            const((H, H)),
            const((1, H)),
            const((1, 1)),
            const((1, 1)),
            const((1, 2 * H)),
            const((1, 2 * H)),
            const((P, H)),
            const((OV, H)),
            const((NA, H)),
            const((2, H)),
            const((1, H)),
            const((1, H)),
            const((1, H)),
        ],
        out_specs=pl.BlockSpec((BB, 3 + S, H), lambda i: (i, 0, 0)),
        out_shape=jax.ShapeDtypeStruct((B, 3 + S, H), jnp.float32),
    )(gath, vals, times, poss, ons, age_i, gen_i, W1, b1, W2, b2,
      lw, lb, pw2, pb2, pos_tab, on_tab, age_tab, gen_tab, task_tab, g_, beta_)


def kernel(label_ids, value_ids, time_ids, on_ids, position_ids, token_type,
           age_ids, gender_ids, task_token, proc_table, med_table, chart_table,
           W1, b1, W2, b2, t2v_lw, t2v_lb, t2v_pw, t2v_pb,
           on_table, pos_table, age_table, gender_table, task_table, ln_g, ln_b):
    B, S = label_ids.shape
    H = proc_table.shape[1]
    V = proc_table.shape[0]
    N = B * S
    lab = label_ids.reshape(N).astype(jnp.int32)
    typ = token_type.reshape(N).astype(jnp.int32)
    stk = jnp.concatenate(
        [proc_table, med_table, chart_table, jnp.zeros((8, H), jnp.float32)], axis=0)
    gath = _sc_label_gather(lab, typ, stk, V).reshape(B, S, H)
    pw_pad = jnp.concatenate([jnp.zeros((1, 1), jnp.float32), t2v_pw], axis=1)
    pb_pad = jnp.concatenate([jnp.zeros((1,), jnp.float32), t2v_pb]).reshape(1, H)
    pw2 = jnp.concatenate([pw_pad, pw_pad], axis=1)
    pb2 = jnp.concatenate([pb_pad, pb_pad], axis=1)
    on_pad = jnp.concatenate(
        [on_table, jnp.zeros((16 - on_table.shape[0], H), jnp.float32)], axis=0)
    return _tc_dense(
        gath,
        value_ids,
        time_ids,
        position_ids.astype(jnp.float32),
        on_ids.astype(jnp.float32),
        age_ids.astype(jnp.int32),
        gender_ids.astype(jnp.int32),
        W1, b1.reshape(1, H), W2, b2.reshape(1, H),
        t2v_lw, t2v_lb.reshape(1, 1), pw2, pb2, pos_table,
        on_pad, age_table, gender_table, task_table,
        ln_g.reshape(1, H), ln_b.reshape(1, H))


# speedup vs baseline: 1.3367x; 1.1831x over previous
import functools

import numpy as np
import jax
import jax.numpy as jnp
from jax import lax
from jax.experimental import pallas as pl
from jax.experimental.pallas import tpu as pltpu
from jax.experimental.pallas import tpu_sc as plsc

_C = 128
_R = 5


def _sc_label_gather(lab, typ, stk, V):
    N = lab.shape[0]
    H = stk.shape[1]
    zidx = 3 * V
    mesh = plsc.VectorSubcoreMesh(
        core_axis_name="c", subcore_axis_name="s", num_cores=2, num_subcores=16
    )
    nw = 32
    per_w = N // nw
    n_chunks = per_w // _C
    n_waves = n_chunks // _R

    @functools.partial(
        pl.kernel,
        out_type=jax.ShapeDtypeStruct((N, H), jnp.bfloat16),
        mesh=mesh,
        compiler_params=pltpu.CompilerParams(
            use_tc_tiling_on_sc=False, disable_bounds_checks=True),
        scratch_types=[
            pltpu.VMEM((per_w,), jnp.int32),
            pltpu.VMEM((per_w,), jnp.int32),
        ]
        + [pltpu.VMEM((_C,), jnp.int32) for _ in range(_R)]
        + [pltpu.VMEM((_C, H), jnp.bfloat16) for _ in range(_R)]
        + [pltpu.SemaphoreType.DMA, pltpu.SemaphoreType.DMA],
    )
    def k(lab_hbm, typ_hbm, stk_hbm, out_hbm, lab_v, typ_v, *rest):
        idx_bufs = rest[:_R]
        row_bufs = rest[_R:2 * _R]
        gsem, wsem = rest[2 * _R], rest[2 * _R + 1]
        wid = lax.axis_index("s") * 2 + lax.axis_index("c")
        base = wid * per_w
        pltpu.sync_copy(lab_hbm.at[pl.ds(base, per_w)], lab_v)
        pltpu.sync_copy(typ_hbm.at[pl.ds(base, per_w)], typ_v)

        def wave(w, carry):
            g0 = w * _R
            gathers = []
            for b in range(_R):
                tok = (g0 + b) * _C
                for r in range(_C // 16):
                    sl_src = pl.ds(tok + r * 16, 16)
                    sl_dst = pl.ds(r * 16, 16)
                    t = typ_v[sl_src]
                    l = lab_v[sl_src]
                    z = jnp.full((16,), zidx, jnp.int32)
                    idx_bufs[b][sl_dst] = jnp.where(t == 0, z, l + (t - 1) * V)
                gathers.append(
                    pltpu.async_copy(stk_hbm.at[idx_bufs[b]], row_bufs[b], gsem))
            for cp in gathers:
                cp.wait()
            writes = []
            for b in range(_R):
                tok = (g0 + b) * _C
                writes.append(
                    pltpu.async_copy(row_bufs[b], out_hbm.at[pl.ds(base + tok, _C)],
                                     wsem))
            for cp in writes:
                cp.wait()
            return carry

        lax.fori_loop(0, n_waves, wave, 0)

    return k(lab, typ, stk)


def _tc_dense(gath, vals, times, poss, ons, age_i, gen_i,
              W1, b1, W2, b2, lw, lb, pw2, pb2, pos_tab,
              on_tab, age_tab, gen_tab, task_tab, g_, beta_):
    B, S, H = gath.shape
    BB = 8
    G = B // BB
    OV = on_tab.shape[0]
    NA = age_tab.shape[0]
    P = pos_tab.shape[0]

    def body(gath_ref, val_ref, time_ref, pos_ref, on_ref,
             age_ref, gen_ref, W1_ref, b1_ref, W2_ref, b2_ref,
             lw_ref, lb_ref, pw2_ref, pb2_ref, postab_ref,
             ontab_ref, agetab_ref, gentab_ref, tasktab_ref,
             g_ref, beta_ref, out_ref):
        def norm(x):
            mu = jnp.mean(x, axis=-1, keepdims=True)
            var = jnp.mean((x - mu) ** 2, axis=-1, keepdims=True)
            return (x - mu) / jnp.sqrt(var + 1e-12) * g_ref[...] + beta_ref[...]

        vT = jnp.transpose(val_ref[...])
        tT = jnp.transpose(time_ref[...])
        pT = jnp.transpose(pos_ref[...])
        oT = jnp.transpose(on_ref[...])
        lane = lax.broadcasted_iota(jnp.int32, (S, H), 1)
        lane2 = lax.broadcasted_iota(jnp.int32, (S, 2 * H), 1)
        ohiota = lax.broadcasted_iota(jnp.int32, (S, OV), 1).astype(jnp.float32)
        phiota = lax.broadcasted_iota(jnp.int32, (S, P), 1).astype(jnp.float32)
        task_n = norm(tasktab_ref[...]).reshape(1, 1, H)
        out_ref[:, 0:1, :] = jnp.broadcast_to(task_n, (BB, 1, H))
        tes = []
        for r in range(0, BB, 2):
            tt0 = tT[:, r:r + 1]
            tt1 = tT[:, r + 1:r + 2]
            tt2 = jnp.where(lane2 < H, tt0, tt1)
            sv = jnp.sin(tt2 * pw2_ref[...] + pb2_ref[...])
            tes.append((tt0, sv[:, 0:H]))
            tes.append((tt1, sv[:, H:2 * H]))
        for r in range(BB):
            rows = gath_ref[r].astype(jnp.float32)
            v = vT[:, r:r + 1]
            h1 = jnp.maximum(v * W1_ref[...] + b1_ref[...], 0.0)
            ve = jnp.dot(h1, W2_ref[...],
                         preferred_element_type=jnp.float32) + b2_ref[...]
            tt, tsin = tes[r]
            te = jnp.where(lane == 0, tt * lw_ref[...] + lb_ref[...], tsin)
            ph = (phiota == pT[:, r:r + 1]).astype(jnp.float32)
            pe = jnp.dot(ph, postab_ref[...], preferred_element_type=jnp.float32)
            oh = (ohiota == oT[:, r:r + 1]).astype(jnp.float32)
            oe = jnp.dot(oh, ontab_ref[...], preferred_element_type=jnp.float32)
            emb = rows + ve + te + pe + oe
            out_ref[r, 3:, :] = norm(emb)
            aid = age_ref[r, 0]
            am = lax.broadcasted_iota(jnp.int32, (NA, H), 0) == aid
            ae = jnp.sum(jnp.where(am, agetab_ref[...], 0.0), axis=0, keepdims=True)
            gid = gen_ref[r, 0]
            gm = lax.broadcasted_iota(jnp.int32, (2, H), 0) == gid
            ge = jnp.sum(jnp.where(gm, gentab_ref[...], 0.0), axis=0, keepdims=True)
            out_ref[r, 1:2, :] = norm(ae)
            out_ref[r, 2:3, :] = norm(ge)

    const = lambda shape: pl.BlockSpec(shape, lambda i: tuple(0 for _ in shape))
    return pl.pallas_call(
        body,
        grid=(G,),
        in_specs=[
            pl.BlockSpec((BB, S, H), lambda i: (i, 0, 0)),
            pl.BlockSpec((BB, S), lambda i: (i, 0)),
            pl.BlockSpec((BB, S), lambda i: (i, 0)),
            pl.BlockSpec((BB, S), lambda i: (i, 0)),
            pl.BlockSpec((BB, S), lambda i: (i, 0)),
            pl.BlockSpec((BB, 1), lambda i: (i, 0), memory_space=pltpu.SMEM),
            pl.BlockSpec((BB, 1), lambda i: (i, 0), memory_space=pltpu.SMEM),
            const((1, H)),
            const((1, H)),
            const((H, H)),
            const((1, H)),
            const((1, 1)),
            const((1, 1)),
            const((1, 2 * H)),
            const((1, 2 * H)),
            const((P, H)),
            const((OV, H)),
            const((NA, H)),
            const((2, H)),
            const((1, H)),
            const((1, H)),
            const((1, H)),
        ],
        out_specs=pl.BlockSpec((BB, 3 + S, H), lambda i: (i, 0, 0)),
        out_shape=jax.ShapeDtypeStruct((B, 3 + S, H), jnp.float32),
    )(gath, vals, times, poss, ons, age_i, gen_i, W1, b1, W2, b2,
      lw, lb, pw2, pb2, pos_tab, on_tab, age_tab, gen_tab, task_tab, g_, beta_)


def kernel(label_ids, value_ids, time_ids, on_ids, position_ids, token_type,
           age_ids, gender_ids, task_token, proc_table, med_table, chart_table,
           W1, b1, W2, b2, t2v_lw, t2v_lb, t2v_pw, t2v_pb,
           on_table, pos_table, age_table, gender_table, task_table, ln_g, ln_b):
    B, S = label_ids.shape
    H = proc_table.shape[1]
    V = proc_table.shape[0]
    N = B * S
    lab = label_ids.reshape(N).astype(jnp.int32)
    typ = token_type.reshape(N).astype(jnp.int32)
    stk = jnp.concatenate(
        [proc_table, med_table, chart_table, jnp.zeros((8, H), jnp.float32)],
        axis=0).astype(jnp.bfloat16)
    gath = _sc_label_gather(lab, typ, stk, V).reshape(B, S, H)
    pw_pad = jnp.concatenate([jnp.zeros((1, 1), jnp.float32), t2v_pw], axis=1)
    pb_pad = jnp.concatenate([jnp.zeros((1,), jnp.float32), t2v_pb]).reshape(1, H)
    pw2 = jnp.concatenate([pw_pad, pw_pad], axis=1)
    pb2 = jnp.concatenate([pb_pad, pb_pad], axis=1)
    on_pad = jnp.concatenate(
        [on_table, jnp.zeros((16 - on_table.shape[0], H), jnp.float32)], axis=0)
    return _tc_dense(
        gath,
        value_ids,
        time_ids,
        position_ids.astype(jnp.float32),
        on_ids.astype(jnp.float32),
        age_ids.astype(jnp.int32),
        gender_ids.astype(jnp.int32),
        W1, b1.reshape(1, H), W2, b2.reshape(1, H),
        t2v_lw, t2v_lb.reshape(1, 1), pw2, pb2, pos_table,
        on_pad, age_table, gender_table, task_table,
        ln_g.reshape(1, H), ln_b.reshape(1, H))


# speedup vs baseline: 1.3533x; 1.0125x over previous
import functools

import numpy as np
import jax
import jax.numpy as jnp
from jax import lax
from jax.experimental import pallas as pl
from jax.experimental.pallas import tpu as pltpu
from jax.experimental.pallas import tpu_sc as plsc

_C = 128
_R = 5


def _sc_label_gather(lab, typ, stk, V, hoff, n_half):
    H = stk.shape[1]
    zidx = 3 * V
    mesh = plsc.VectorSubcoreMesh(
        core_axis_name="c", subcore_axis_name="s", num_cores=2, num_subcores=16
    )
    nw = 32
    per_w = n_half // nw
    n_chunks = per_w // _C
    n_waves = n_chunks // _R

    @functools.partial(
        pl.kernel,
        out_type=jax.ShapeDtypeStruct((n_half, H), jnp.bfloat16),
        mesh=mesh,
        compiler_params=pltpu.CompilerParams(
            use_tc_tiling_on_sc=False, disable_bounds_checks=True),
        scratch_types=[
            pltpu.VMEM((per_w,), jnp.int32),
            pltpu.VMEM((per_w,), jnp.int32),
        ]
        + [pltpu.VMEM((_C,), jnp.int32) for _ in range(_R)]
        + [pltpu.VMEM((_C, H), jnp.bfloat16) for _ in range(_R)]
        + [pltpu.SemaphoreType.DMA, pltpu.SemaphoreType.DMA],
    )
    def k(lab_hbm, typ_hbm, stk_hbm, out_hbm, lab_v, typ_v, *rest):
        idx_bufs = rest[:_R]
        row_bufs = rest[_R:2 * _R]
        gsem, wsem = rest[2 * _R], rest[2 * _R + 1]
        wid = lax.axis_index("s") * 2 + lax.axis_index("c")
        base = wid * per_w
        pltpu.sync_copy(lab_hbm.at[pl.ds(hoff + base, per_w)], lab_v)
        pltpu.sync_copy(typ_hbm.at[pl.ds(hoff + base, per_w)], typ_v)

        def wave(w, carry):
            g0 = w * _R
            gathers = []
            for b in range(_R):
                tok = (g0 + b) * _C
                for r in range(_C // 16):
                    sl_src = pl.ds(tok + r * 16, 16)
                    sl_dst = pl.ds(r * 16, 16)
                    t = typ_v[sl_src]
                    l = lab_v[sl_src]
                    z = jnp.full((16,), zidx, jnp.int32)
                    idx_bufs[b][sl_dst] = jnp.where(t == 0, z, l + (t - 1) * V)
                gathers.append(
                    pltpu.async_copy(stk_hbm.at[idx_bufs[b]], row_bufs[b], gsem))
            for cp in gathers:
                cp.wait()
            writes = []
            for b in range(_R):
                tok = (g0 + b) * _C
                writes.append(
                    pltpu.async_copy(row_bufs[b], out_hbm.at[pl.ds(base + tok, _C)],
                                     wsem))
            for cp in writes:
                cp.wait()
            return carry

        lax.fori_loop(0, n_waves, wave, 0)

    return k(lab, typ, stk)


def _tc_dense(gath, vals, times, poss, ons, age_i, gen_i,
              W1, b1, W2, b2, lw, lb, pw2, pb2, pos_tab,
              on_tab, age_tab, gen_tab, task_tab, g_, beta_,
              B_total, goff, out_prev):
    Bh, S, H = gath.shape
    BB = 8
    G = Bh // BB
    OV = on_tab.shape[0]
    NA = age_tab.shape[0]
    P = pos_tab.shape[0]

    def body(*refs):
        if out_prev is not None:
            refs = refs[1:]
        (gath_ref, val_ref, time_ref, pos_ref, on_ref,
         age_ref, gen_ref, W1_ref, b1_ref, W2_ref, b2_ref,
         lw_ref, lb_ref, pw2_ref, pb2_ref, postab_ref,
         ontab_ref, agetab_ref, gentab_ref, tasktab_ref,
         g_ref, beta_ref, out_ref) = refs
        def norm(x):
            mu = jnp.mean(x, axis=-1, keepdims=True)
            var = jnp.mean((x - mu) ** 2, axis=-1, keepdims=True)
            return (x - mu) / jnp.sqrt(var + 1e-12) * g_ref[...] + beta_ref[...]

        vT = jnp.transpose(val_ref[...])
        tT = jnp.transpose(time_ref[...])
        pT = jnp.transpose(pos_ref[...])
        oT = jnp.transpose(on_ref[...])
        lane = lax.broadcasted_iota(jnp.int32, (S, H), 1)
        lane2 = lax.broadcasted_iota(jnp.int32, (S, 2 * H), 1)
        ohiota = lax.broadcasted_iota(jnp.int32, (S, OV), 1).astype(jnp.float32)
        phiota = lax.broadcasted_iota(jnp.int32, (S, P), 1).astype(jnp.float32)
        task_n = norm(tasktab_ref[...]).reshape(1, 1, H)
        out_ref[:, 0:1, :] = jnp.broadcast_to(task_n, (BB, 1, H))
        tes = []
        for r in range(0, BB, 2):
            tt0 = tT[:, r:r + 1]
            tt1 = tT[:, r + 1:r + 2]
            tt2 = jnp.where(lane2 < H, tt0, tt1)
            sv = jnp.sin(tt2 * pw2_ref[...] + pb2_ref[...])
            tes.append((tt0, sv[:, 0:H]))
            tes.append((tt1, sv[:, H:2 * H]))
        for r in range(BB):
            rows = gath_ref[r].astype(jnp.float32)
            v = vT[:, r:r + 1]
            h1 = jnp.maximum(v * W1_ref[...] + b1_ref[...], 0.0)
            ve = jnp.dot(h1, W2_ref[...],
                         preferred_element_type=jnp.float32) + b2_ref[...]
            tt, tsin = tes[r]
            te = jnp.where(lane == 0, tt * lw_ref[...] + lb_ref[...], tsin)
            ph = (phiota == pT[:, r:r + 1]).astype(jnp.float32)
            pe = jnp.dot(ph, postab_ref[...], preferred_element_type=jnp.float32)
            oh = (ohiota == oT[:, r:r + 1]).astype(jnp.float32)
            oe = jnp.dot(oh, ontab_ref[...], preferred_element_type=jnp.float32)
            emb = rows + ve + te + pe + oe
            out_ref[r, 3:, :] = norm(emb)
            aid = age_ref[r, 0]
            am = lax.broadcasted_iota(jnp.int32, (NA, H), 0) == aid
            ae = jnp.sum(jnp.where(am, agetab_ref[...], 0.0), axis=0, keepdims=True)
            gid = gen_ref[r, 0]
            gm = lax.broadcasted_iota(jnp.int32, (2, H), 0) == gid
            ge = jnp.sum(jnp.where(gm, gentab_ref[...], 0.0), axis=0, keepdims=True)
            out_ref[r, 1:2, :] = norm(ae)
            out_ref[r, 2:3, :] = norm(ge)

    const = lambda shape: pl.BlockSpec(shape, lambda i: tuple(0 for _ in shape))
    in_specs = [
            pl.BlockSpec((BB, S, H), lambda i: (i, 0, 0)),
            pl.BlockSpec((BB, S), lambda i: (i + goff, 0)),
            pl.BlockSpec((BB, S), lambda i: (i + goff, 0)),
            pl.BlockSpec((BB, S), lambda i: (i + goff, 0)),
            pl.BlockSpec((BB, S), lambda i: (i + goff, 0)),
            pl.BlockSpec((BB, 1), lambda i: (i + goff, 0),
                         memory_space=pltpu.SMEM),
            pl.BlockSpec((BB, 1), lambda i: (i + goff, 0),
                         memory_space=pltpu.SMEM),
            const((1, H)),
            const((1, H)),
            const((H, H)),
            const((1, H)),
            const((1, 1)),
            const((1, 1)),
            const((1, 2 * H)),
            const((1, 2 * H)),
            const((P, H)),
            const((OV, H)),
            const((NA, H)),
            const((2, H)),
            const((1, H)),
            const((1, H)),
            const((1, H)),
        ]
    args = [gath, vals, times, poss, ons, age_i, gen_i, W1, b1, W2, b2,
            lw, lb, pw2, pb2, pos_tab, on_tab, age_tab, gen_tab, task_tab,
            g_, beta_]
    aliases = {}
    if out_prev is not None:
        in_specs = [pl.BlockSpec(memory_space=pl.ANY)] + in_specs
        args = [out_prev] + args
        aliases = {0: 0}
    return pl.pallas_call(
        body,
        grid=(G,),
        in_specs=in_specs,
        out_specs=pl.BlockSpec((BB, 3 + S, H), lambda i: (i + goff, 0, 0)),
        out_shape=jax.ShapeDtypeStruct((B_total, 3 + S, H), jnp.float32),
        input_output_aliases=aliases,
    )(*args)


def kernel(label_ids, value_ids, time_ids, on_ids, position_ids, token_type,
           age_ids, gender_ids, task_token, proc_table, med_table, chart_table,
           W1, b1, W2, b2, t2v_lw, t2v_lb, t2v_pw, t2v_pb,
           on_table, pos_table, age_table, gender_table, task_table, ln_g, ln_b):
    B, S = label_ids.shape
    H = proc_table.shape[1]
    V = proc_table.shape[0]
    N = B * S
    lab = label_ids.reshape(N).astype(jnp.int32)
    typ = token_type.reshape(N).astype(jnp.int32)
    stk = jnp.concatenate(
        [proc_table, med_table, chart_table, jnp.zeros((8, H), jnp.float32)],
        axis=0).astype(jnp.bfloat16)
    Nh = N // 2
    Bh = B // 2
    gath_a = _sc_label_gather(lab, typ, stk, V, 0, Nh).reshape(Bh, S, H)
    gath_b = _sc_label_gather(lab, typ, stk, V, Nh, Nh).reshape(Bh, S, H)
    pw_pad = jnp.concatenate([jnp.zeros((1, 1), jnp.float32), t2v_pw], axis=1)
    pb_pad = jnp.concatenate([jnp.zeros((1,), jnp.float32), t2v_pb]).reshape(1, H)
    pw2 = jnp.concatenate([pw_pad, pw_pad], axis=1)
    pb2 = jnp.concatenate([pb_pad, pb_pad], axis=1)
    on_pad = jnp.concatenate(
        [on_table, jnp.zeros((16 - on_table.shape[0], H), jnp.float32)], axis=0)
    common = (value_ids, time_ids,
              position_ids.astype(jnp.float32),
              on_ids.astype(jnp.float32),
              age_ids.astype(jnp.int32),
              gender_ids.astype(jnp.int32),
              W1, b1.reshape(1, H), W2, b2.reshape(1, H),
              t2v_lw, t2v_lb.reshape(1, 1), pw2, pb2, pos_table,
              on_pad, age_table, gender_table, task_table,
              ln_g.reshape(1, H), ln_b.reshape(1, H))
    out_a = _tc_dense(gath_a, *common, B, 0, None)
    return _tc_dense(gath_b, *common, B, Bh // 8, out_a)


# speedup vs baseline: 1.3559x; 1.0019x over previous
import functools

import numpy as np
import jax
import jax.numpy as jnp
from jax import lax
from jax.experimental import pallas as pl
from jax.experimental.pallas import tpu as pltpu
from jax.experimental.pallas import tpu_sc as plsc

_C = 128
_R = 5


def _sc_label_gather(cpk, stk, V, hoff, n_half):
    H = stk.shape[1]
    zidx = 3 * V
    mesh = plsc.VectorSubcoreMesh(
        core_axis_name="c", subcore_axis_name="s", num_cores=2, num_subcores=16
    )
    nw = 32
    per_w = n_half // nw
    n_chunks = per_w // _C
    n_waves = n_chunks // _R

    @functools.partial(
        pl.kernel,
        out_type=jax.ShapeDtypeStruct((n_half, H), jnp.bfloat16),
        mesh=mesh,
        compiler_params=pltpu.CompilerParams(
            use_tc_tiling_on_sc=False, disable_bounds_checks=True),
        scratch_types=[
            pltpu.VMEM((per_w,), jnp.int32),
        ]
        + [pltpu.VMEM((_C,), jnp.int32) for _ in range(_R)]
        + [pltpu.VMEM((_C, H), jnp.bfloat16) for _ in range(_R)]
        + [pltpu.SemaphoreType.DMA, pltpu.SemaphoreType.DMA],
    )
    def k(cpk_hbm, stk_hbm, out_hbm, cpk_v, *rest):
        idx_bufs = rest[:_R]
        row_bufs = rest[_R:2 * _R]
        gsem, wsem = rest[2 * _R], rest[2 * _R + 1]
        wid = lax.axis_index("s") * 2 + lax.axis_index("c")
        base = wid * per_w
        pltpu.sync_copy(cpk_hbm.at[pl.ds(hoff + base, per_w)], cpk_v)

        def wave(w, carry):
            g0 = w * _R
            gathers = []
            for b in range(_R):
                tok = (g0 + b) * _C
                for r in range(_C // 16):
                    sl_src = pl.ds(tok + r * 16, 16)
                    sl_dst = pl.ds(r * 16, 16)
                    c = cpk_v[sl_src]
                    t = c & 3
                    l = lax.shift_right_logical(c, 2)
                    z = jnp.full((16,), zidx, jnp.int32)
                    idx_bufs[b][sl_dst] = jnp.where(t == 0, z, l + (t - 1) * V)
                gathers.append(
                    pltpu.async_copy(stk_hbm.at[idx_bufs[b]], row_bufs[b], gsem))
            for cp in gathers:
                cp.wait()
            writes = []
            for b in range(_R):
                tok = (g0 + b) * _C
                writes.append(
                    pltpu.async_copy(row_bufs[b], out_hbm.at[pl.ds(base + tok, _C)],
                                     wsem))
            for cp in writes:
                cp.wait()
            return carry

        lax.fori_loop(0, n_waves, wave, 0)

    return k(cpk, stk)


def _tc_dense(gath, vals, times, poss, ons, age_i, gen_i,
              W1, b1, W2, b2, lw, lb, pw2, pb2, pos_tab,
              on_tab, age_tab, gen_tab, task_tab, g_, beta_,
              B_total, goff, out_prev):
    Nh, H = gath.shape
    BB = 8
    S = 200
    Bh = Nh // S
    G = Bh // BB
    OV = on_tab.shape[0]
    NA = age_tab.shape[0]
    P = pos_tab.shape[0]

    def body(*refs):
        if out_prev is not None:
            refs = refs[1:]
        (gath_ref, val_ref, time_ref, pos_ref, on_ref,
         age_ref, gen_ref, W1_ref, b1_ref, W2_ref, b2_ref,
         lw_ref, lb_ref, pw2_ref, pb2_ref, postab_ref,
         ontab_ref, agetab_ref, gentab_ref, tasktab_ref,
         g_ref, beta_ref, out_ref) = refs
        def norm(x):
            mu = jnp.mean(x, axis=-1, keepdims=True)
            var = jnp.mean((x - mu) ** 2, axis=-1, keepdims=True)
            return (x - mu) / jnp.sqrt(var + 1e-12) * g_ref[...] + beta_ref[...]

        vT = jnp.transpose(val_ref[...])
        tT = jnp.transpose(time_ref[...])
        pT = jnp.transpose(pos_ref[...])
        oT = jnp.transpose(on_ref[...])
        lane = lax.broadcasted_iota(jnp.int32, (S, H), 1)
        lane2 = lax.broadcasted_iota(jnp.int32, (S, 2 * H), 1)
        ohiota = lax.broadcasted_iota(jnp.int32, (S, OV), 1).astype(jnp.float32)
        phiota = lax.broadcasted_iota(jnp.int32, (S, P), 1).astype(jnp.float32)
        task_n = norm(tasktab_ref[...]).reshape(1, 1, H)
        out_ref[:, 0:1, :] = jnp.broadcast_to(task_n, (BB, 1, H))
        tes = []
        for r in range(0, BB, 2):
            tt0 = tT[:, r:r + 1]
            tt1 = tT[:, r + 1:r + 2]
            tt2 = jnp.where(lane2 < H, tt0, tt1)
            sv = jnp.sin(tt2 * pw2_ref[...] + pb2_ref[...])
            tes.append((tt0, sv[:, 0:H]))
            tes.append((tt1, sv[:, H:2 * H]))
        for r in range(BB):
            rows = gath_ref[pl.ds(r * S, S), :].astype(jnp.float32)
            v = vT[:, r:r + 1]
            h1 = jnp.maximum(v * W1_ref[...] + b1_ref[...], 0.0)
            ve = jnp.dot(h1, W2_ref[...],
                         preferred_element_type=jnp.float32) + b2_ref[...]
            tt, tsin = tes[r]
            te = jnp.where(lane == 0, tt * lw_ref[...] + lb_ref[...], tsin)
            ph = (phiota == pT[:, r:r + 1]).astype(jnp.float32)
            pe = jnp.dot(ph, postab_ref[...], preferred_element_type=jnp.float32)
            oh = (ohiota == oT[:, r:r + 1]).astype(jnp.float32)
            oe = jnp.dot(oh, ontab_ref[...], preferred_element_type=jnp.float32)
            emb = rows + ve + te + pe + oe
            out_ref[r, 3:, :] = norm(emb)
            aid = age_ref[r, 0]
            am = lax.broadcasted_iota(jnp.int32, (NA, H), 0) == aid
            ae = jnp.sum(jnp.where(am, agetab_ref[...], 0.0), axis=0, keepdims=True)
            gid = gen_ref[r, 0]
            gm = lax.broadcasted_iota(jnp.int32, (2, H), 0) == gid
            ge = jnp.sum(jnp.where(gm, gentab_ref[...], 0.0), axis=0, keepdims=True)
            out_ref[r, 1:2, :] = norm(ae)
            out_ref[r, 2:3, :] = norm(ge)

    const = lambda shape: pl.BlockSpec(shape, lambda i: tuple(0 for _ in shape))
    in_specs = [
            pl.BlockSpec((BB * S, H), lambda i: (i, 0)),
            pl.BlockSpec((BB, S), lambda i: (i + goff, 0)),
            pl.BlockSpec((BB, S), lambda i: (i + goff, 0)),
            pl.BlockSpec((BB, S), lambda i: (i + goff, 0)),
            pl.BlockSpec((BB, S), lambda i: (i + goff, 0)),
            pl.BlockSpec((BB, 1), lambda i: (i + goff, 0),
                         memory_space=pltpu.SMEM),
            pl.BlockSpec((BB, 1), lambda i: (i + goff, 0),
                         memory_space=pltpu.SMEM),
            const((1, H)),
            const((1, H)),
            const((H, H)),
            const((1, H)),
            const((1, 1)),
            const((1, 1)),
            const((1, 2 * H)),
            const((1, 2 * H)),
            const((P, H)),
            const((OV, H)),
            const((NA, H)),
            const((2, H)),
            const((1, H)),
            const((1, H)),
            const((1, H)),
        ]
    args = [gath, vals, times, poss, ons, age_i, gen_i, W1, b1, W2, b2,
            lw, lb, pw2, pb2, pos_tab, on_tab, age_tab, gen_tab, task_tab,
            g_, beta_]
    aliases = {}
    if out_prev is not None:
        in_specs = [pl.BlockSpec(memory_space=pl.ANY)] + in_specs
        args = [out_prev] + args
        aliases = {0: 0}
    return pl.pallas_call(
        body,
        grid=(G,),
        in_specs=in_specs,
        out_specs=pl.BlockSpec((BB, 3 + S, H), lambda i: (i + goff, 0, 0)),
        out_shape=jax.ShapeDtypeStruct((B_total, 3 + S, H), jnp.float32),
        input_output_aliases=aliases,
    )(*args)


def kernel(label_ids, value_ids, time_ids, on_ids, position_ids, token_type,
           age_ids, gender_ids, task_token, proc_table, med_table, chart_table,
           W1, b1, W2, b2, t2v_lw, t2v_lb, t2v_pw, t2v_pb,
           on_table, pos_table, age_table, gender_table, task_table, ln_g, ln_b):
    B, S = label_ids.shape
    H = proc_table.shape[1]
    V = proc_table.shape[0]
    N = B * S
    cpk = (label_ids.astype(jnp.int32) * 4
           + token_type.astype(jnp.int32)).reshape(N)
    stk = jnp.concatenate(
        [proc_table, med_table, chart_table, jnp.zeros((8, H), jnp.float32)],
        axis=0).astype(jnp.bfloat16)
    Nh = N // 2
    Bh = B // 2
    gath_a = _sc_label_gather(cpk, stk, V, 0, Nh)
    gath_b = _sc_label_gather(cpk, stk, V, Nh, Nh)
    pw_pad = jnp.concatenate([jnp.zeros((1, 1), jnp.float32), t2v_pw], axis=1)
    pb_pad = jnp.concatenate([jnp.zeros((1,), jnp.float32), t2v_pb]).reshape(1, H)
    pw2 = jnp.concatenate([pw_pad, pw_pad], axis=1)
    pb2 = jnp.concatenate([pb_pad, pb_pad], axis=1)
    on_pad = jnp.concatenate(
        [on_table, jnp.zeros((16 - on_table.shape[0], H), jnp.float32)], axis=0)
    common = (value_ids, time_ids,
              position_ids.astype(jnp.float32),
              on_ids.astype(jnp.float32),
              age_ids.astype(jnp.int32),
              gender_ids.astype(jnp.int32),
              W1, b1.reshape(1, H), W2, b2.reshape(1, H),
              t2v_lw, t2v_lb.reshape(1, 1), pw2, pb2, pos_table,
              on_pad, age_table, gender_table, task_table,
              ln_g.reshape(1, H), ln_b.reshape(1, H))
    out_a = _tc_dense(gath_a, *common, B, 0, None)
    return _tc_dense(gath_b, *common, B, Bh // 8, out_a)
